# baseline (speedup 1.0000x reference)
import jax
import jax.numpy as jnp
from jax.experimental import pallas as pl

HID = 32
HEADS = 4
OUT = HID * HEADS


def _lin(x, p):
    return x @ p[0] + p[1]


def _ln(x, p, eps=1e-5):
    m = jnp.mean(x, -1, keepdims=True)
    v = jnp.var(x, -1, keepdims=True)
    return (x - m) / jnp.sqrt(v + eps) * p[0] + p[1]


def _conv(cp, x, src, dst, e, n):
    nd = x.shape[1]
    Wa = cp['a1'][0]
    Ad, As, Ae = Wa[:nd], Wa[nd:2 * nd], Wa[2 * nd:]
    ai = x @ Ad
    aj = x @ As
    h = jax.nn.relu(ai[dst] + aj[src] + e @ Ae + cp['a1'][1])
    s = _lin(h, cp['a2'])
    ex = jnp.exp(s)
    Wm = cp['m1'][0]
    Mx, Me = Wm[:nd], Wm[nd:]
    mx = x @ Mx
    mp = jax.nn.relu(mx[src] + e @ Me + cp['m1'][1])
    m = _lin(mp, cp['m2'])
    wm = (m.reshape(-1, HEADS, HID) * ex[:, :, None]).reshape(-1, OUT)
    agg = jax.ops.segment_sum(wm, dst, num_segments=n)
    den = jax.ops.segment_sum(ex, dst, num_segments=n)
    den = jnp.repeat(den + 1e-16, HID, axis=1)
    return agg / den


def _eup(up, x, src, dst, e):
    W = up['w1'][0]
    nd = x.shape[1]
    Wd, Ws, We = W[:nd], W[nd:2 * nd], W[2 * nd:]
    g = (x @ Wd)[dst] + (x @ Ws)[src] + e @ We + up['w1'][1]
    delta = _lin(jax.nn.relu(g), up['w2'])
    return _ln(e + delta, up['ln'])


def kernel(x_in, edge_index, edge_attr, params):
    P = params
    n = x_in.shape[0]
    perm = jnp.argsort(edge_index[1])
    src = edge_index[0][perm]
    dst = edge_index[1][perm]
    e_in = edge_attr[perm]

    invalid = x_in[:, 0] == -999.0
    x_clean = jnp.where(invalid[:, None], P['dummy'][None, :], x_in)
    x0 = _lin(jax.nn.relu(_lin(x_clean, P['ne1'])), P['ne2'])
    e0 = _lin(jax.nn.relu(_lin(e_in, P['ee1'])), P['ee2'])

    x1 = _conv(P['conv1'], x0, src, dst, e0, n) + _lin(x0, P['skip0'])
    x1 = jax.nn.relu(_ln(x1, P['ln1']))
    e1 = _eup(P['eup1'], x1, src, dst, e0)
    x2 = _conv(P['conv2'], x1, src, dst, e1, n) + x1
    x2 = jax.nn.relu(_ln(x2, P['ln2']))
    e2 = _eup(P['eup2'], x2, src, dst, e1)
    x3 = _conv(P['conv3'], x2, src, dst, e2, n) + x2
    x3 = jax.nn.relu(_ln(x3, P['ln3']))
    x_out = x3 + _lin(x1, P['skipL'])

    h = jax.nn.relu(_lin(x_out, P['nh1']))
    h = jax.nn.relu(_lin(h, P['nh2']))
    h = jax.nn.relu(_lin(h, P['nh3']))
    node_logits = _lin(h, P['nh4'])
    node_probs = jax.nn.softmax(node_logits, -1)

    We1 = P['eh1'][0]
    Hs, Hd, He = We1[:OUT], We1[OUT:2 * OUT], We1[2 * OUT:]
    g = jax.nn.relu((x_out @ Hd)[dst] + (x_out @ Hs)[src] + e2 @ He + P['eh1'][1])
    g = jax.nn.relu(_lin(g, P['eh2']))
    g = jax.nn.relu(_lin(g, P['eh3']))
    el_sorted = _lin(g, P['eh4'])
    edge_logits = jnp.zeros_like(el_sorted).at[perm].set(el_sorted)
    edge_probs = jax.nn.sigmoid(edge_logits)
    return (node_logits, edge_logits, node_probs, edge_probs)


# speedup vs baseline: 1.0143x; 1.0143x over previous
import jax
import jax.numpy as jnp
from jax.experimental import pallas as pl

HID = 32
HEADS = 4
OUT = HID * HEADS


def _lin(x, p):
    return x @ p[0] + p[1]


def _ln(x, p, eps=1e-5):
    m = jnp.mean(x, -1, keepdims=True)
    v = jnp.var(x, -1, keepdims=True)
    return (x - m) / jnp.sqrt(v + eps) * p[0] + p[1]


def _conv(cp, x, src, dst, e, n):
    nd = x.shape[1]
    Wa = cp['a1'][0]
    Ad, As, Ae = Wa[:nd], Wa[nd:2 * nd], Wa[2 * nd:]
    ai = x @ Ad
    aj = x @ As
    h = jax.nn.relu(ai[dst] + aj[src] + e @ Ae + cp['a1'][1])
    s = _lin(h, cp['a2'])
    ex = jnp.exp(s)
    Wm = cp['m1'][0]
    Mx, Me = Wm[:nd], Wm[nd:]
    mx = x @ Mx
    mp = jax.nn.relu(mx[src] + e @ Me + cp['m1'][1])
    m = _lin(mp, cp['m2'])
    wm = (m.reshape(-1, HEADS, HID) * ex[:, :, None]).reshape(-1, OUT)
    agg = jax.ops.segment_sum(wm, dst, num_segments=n, indices_are_sorted=True)
    den = jax.ops.segment_sum(ex, dst, num_segments=n, indices_are_sorted=True)
    den = jnp.repeat(den + 1e-16, HID, axis=1)
    return agg / den


def _eup(up, x, src, dst, e):
    W = up['w1'][0]
    nd = x.shape[1]
    Wd, Ws, We = W[:nd], W[nd:2 * nd], W[2 * nd:]
    g = (x @ Wd)[dst] + (x @ Ws)[src] + e @ We + up['w1'][1]
    delta = _lin(jax.nn.relu(g), up['w2'])
    return _ln(e + delta, up['ln'])


def kernel(x_in, edge_index, edge_attr, params):
    P = params
    n = x_in.shape[0]
    perm = jnp.argsort(edge_index[1])
    src = edge_index[0][perm]
    dst = edge_index[1][perm]
    e_in = edge_attr[perm]

    invalid = x_in[:, 0] == -999.0
    x_clean = jnp.where(invalid[:, None], P['dummy'][None, :], x_in)
    x0 = _lin(jax.nn.relu(_lin(x_clean, P['ne1'])), P['ne2'])
    e0 = _lin(jax.nn.relu(_lin(e_in, P['ee1'])), P['ee2'])

    x1 = _conv(P['conv1'], x0, src, dst, e0, n) + _lin(x0, P['skip0'])
    x1 = jax.nn.relu(_ln(x1, P['ln1']))
    e1 = _eup(P['eup1'], x1, src, dst, e0)
    x2 = _conv(P['conv2'], x1, src, dst, e1, n) + x1
    x2 = jax.nn.relu(_ln(x2, P['ln2']))
    e2 = _eup(P['eup2'], x2, src, dst, e1)
    x3 = _conv(P['conv3'], x2, src, dst, e2, n) + x2
    x3 = jax.nn.relu(_ln(x3, P['ln3']))
    x_out = x3 + _lin(x1, P['skipL'])

    h = jax.nn.relu(_lin(x_out, P['nh1']))
    h = jax.nn.relu(_lin(h, P['nh2']))
    h = jax.nn.relu(_lin(h, P['nh3']))
    node_logits = _lin(h, P['nh4'])
    node_probs = jax.nn.softmax(node_logits, -1)

    We1 = P['eh1'][0]
    Hs, Hd, He = We1[:OUT], We1[OUT:2 * OUT], We1[2 * OUT:]
    g = jax.nn.relu((x_out @ Hd)[dst] + (x_out @ Hs)[src] + e2 @ He + P['eh1'][1])
    g = jax.nn.relu(_lin(g, P['eh2']))
    g = jax.nn.relu(_lin(g, P['eh3']))
    el_sorted = _lin(g, P['eh4'])
    edge_logits = jnp.zeros_like(el_sorted).at[perm].set(el_sorted)
    edge_probs = jax.nn.sigmoid(edge_logits)
    return (node_logits, edge_logits, node_probs, edge_probs)


# speedup vs baseline: 1.7743x; 1.7493x over previous
import functools

import jax
import jax.numpy as jnp
from jax import lax
from jax.experimental import pallas as pl
from jax.experimental.pallas import tpu as pltpu
from jax.experimental.pallas import tpu_sc as plsc

N = 10000
E = 320000
EPAD = 323584
NODE_IN = 128
HID = 32
HEADS = 4
OUT = 128
NC = 2
NW = 32
EW = EPAD // NW
CH = 128
NCHUNK = EW // CH
RPW = 313
NPAD = NW * RPW
TE = 512
GE = EPAD // TE
TN = 400
GN = N // TN
NEG = -2147483648

_mesh = plsc.VectorSubcoreMesh(core_axis_name="c", subcore_axis_name="s")


def _wid():
    return lax.axis_index("s") * NC + lax.axis_index("c")


def _sc_gather2(tbl_d, tbl_s, idx_d, idx_s, scat_vals=None, scat_idx=None):
    do_scat = scat_vals is not None

    out_type = [jax.ShapeDtypeStruct((EPAD, OUT), jnp.float32),
                jax.ShapeDtypeStruct((EPAD, OUT), jnp.float32)]
    if do_scat:
        out_type.append(jax.ShapeDtypeStruct((EPAD, OUT), jnp.float32))
    scratch = [pltpu.VMEM((CH,), jnp.int32), pltpu.VMEM((CH,), jnp.int32),
               pltpu.VMEM((CH, OUT), jnp.float32),
               pltpu.VMEM((CH, OUT), jnp.float32)]
    if do_scat:
        scratch += [pltpu.VMEM((CH,), jnp.int32),
                    pltpu.VMEM((CH, OUT), jnp.float32)]
    scratch.append(pltpu.SemaphoreType.DMA)

    @functools.partial(pl.kernel, out_type=out_type, mesh=_mesh,
                       scratch_types=scratch)
    def k(*refs):
        nin = 4 + (2 if do_scat else 0)
        nout = 2 + (1 if do_scat else 0)
        td, ts, ixd, ixs = refs[0], refs[1], refs[2], refs[3]
        outs = refs[nin:nin + nout]
        scr = refs[nin + nout:]
        ixd_v, ixs_v, bd, bs = scr[0], scr[1], scr[2], scr[3]
        sem = scr[-1]
        base = _wid() * EW

        def chunk(kk, carry):
            off = base + kk * CH
            pltpu.sync_copy(ixd.at[pl.ds(off, CH)], ixd_v)
            pltpu.sync_copy(ixs.at[pl.ds(off, CH)], ixs_v)
            pltpu.async_copy(td.at[ixd_v], bd, sem).wait()
            pltpu.sync_copy(bd, outs[0].at[pl.ds(off, CH)])
            pltpu.async_copy(ts.at[ixs_v], bs, sem).wait()
            pltpu.sync_copy(bs, outs[1].at[pl.ds(off, CH)])
            if do_scat:
                sv, si = refs[4], refs[5]
                pv, vbuf = scr[4], scr[5]
                pltpu.sync_copy(si.at[pl.ds(off, CH)], pv)
                pltpu.sync_copy(sv.at[pl.ds(off, CH)], vbuf)
                pltpu.async_copy(vbuf, outs[2].at[pv], sem).wait()
            return carry

        lax.fori_loop(0, NCHUNK, chunk, 0)

    args = [tbl_d, tbl_s, idx_d, idx_s]
    if do_scat:
        args += [scat_vals, scat_idx]
    return k(*args)


@functools.partial(
    pl.kernel,
    out_type=[jax.ShapeDtypeStruct((NPAD * OUT,), jnp.float32),
              jax.ShapeDtypeStruct((NPAD * 16,), jnp.float32)],
    mesh=_mesh,
    scratch_types=[pltpu.VMEM((CH + 16,), jnp.int32),
                   pltpu.VMEM((CH * OUT,), jnp.float32),
                   pltpu.VMEM((CH * 4 + 16,), jnp.float32),
                   pltpu.VMEM((128,), jnp.int32),
                   pltpu.VMEM((RPW * OUT,), jnp.float32),
                   pltpu.VMEM((RPW * 16,), jnp.float32)],
)
def _sc_segsum(dst, wmf, exf, spans, aggf, denf,
               dbuf, wmbuf, exbuf, spanv, accf, dnf):
    w = _wid()
    lo = w * RPW
    iot = lax.broadcasted_iota(jnp.int32, (16,), 0)

    def zero_acc(k, c):
        accf[pl.ds(k * 16, 16)] = jnp.zeros((16,), jnp.float32)
        return c

    def zero_den(k, c):
        dnf[pl.ds(k * 16, 16)] = jnp.zeros((16,), jnp.float32)
        return c

    lax.fori_loop(0, RPW * OUT // 16, zero_acc, 0)
    lax.fori_loop(0, RPW, zero_den, 0)

    pltpu.sync_copy(spans.at[w], spanv)
    sv = spanv[pl.ds(0, 16)]
    s0 = sv[0]
    s1 = sv[1]
    a0 = (s0 // CH) * CH
    nch = (s1 - a0 + CH - 1) // CH

    zero16 = jnp.zeros((16,), jnp.float32)
    init = ([zero16] * 8, zero16, jnp.int32(-1))

    def flush(accs, dacc, cur):
        @pl.when(cur >= 0)
        def _():
            rb = (cur - lo) * OUT
            for c in range(8):
                accf[pl.ds(rb + c * 16, 16)] = accs[c]
            dnf[pl.ds((cur - lo) * 16, 16)] = dacc

    def chunk(kk, carry):
        off = a0 + kk * CH
        pltpu.sync_copy(dst.at[pl.ds(off, CH)], dbuf.at[pl.ds(0, CH)])
        pltpu.sync_copy(wmf.at[pl.ds(off * OUT, CH * OUT)], wmbuf)
        pltpu.sync_copy(exf.at[pl.ds(off * 4, CH * 4)],
                        exbuf.at[pl.ds(0, CH * 4)])
        p_lo = jnp.maximum(s0 - off, 0)
        p_hi = jnp.minimum(s1 - off, CH)

        def edge(p, ec):
            accs, dacc, cur = ec
            d = dbuf[pl.ds(p, 16)][0]
            change = d != cur

            @pl.when(change)
            def _():
                flush(accs, dacc, cur)

            row = [wmbuf[pl.ds(p * OUT + c * 16, 16)] for c in range(8)]
            exrow = jnp.where(iot < 4, exbuf[pl.ds(p * 4, 16)], 0.0)
            naccs = [jnp.where(change, 0.0, accs[c]) + row[c]
                     for c in range(8)]
            ndacc = jnp.where(change, 0.0, dacc) + exrow
            return (naccs, ndacc, d)

        return lax.fori_loop(p_lo, p_hi, edge, carry)

    accs, dacc, cur = lax.fori_loop(0, nch, chunk, init)
    flush(accs, dacc, cur)
    pltpu.sync_copy(accf, aggf.at[pl.ds(lo * OUT, RPW * OUT)])
    pltpu.sync_copy(dnf, denf.at[pl.ds(lo * 16, RPW * 16)])


def _full(shape):
    return pl.BlockSpec(shape, lambda i: tuple(0 for _ in shape))


def _rows(tile, w):
    return pl.BlockSpec((tile, w), lambda i: (i, 0))


def _ln_in(x, g, b, eps=1e-5):
    m = jnp.mean(x, -1, keepdims=True)
    v = jnp.mean((x - m) ** 2, -1, keepdims=True)
    return (x - m) * jax.lax.rsqrt(v + eps) * g + b


def _tc(body, grid, in_specs, out_specs, out_shapes):
    return pl.pallas_call(body, grid=grid, in_specs=in_specs,
                          out_specs=out_specs, out_shape=out_shapes)


def _r2(b):
    return b.reshape(1, -1)


def _node_pre(x_in, P):
    wne1, bne1 = P['ne1']
    wne2, bne2 = P['ne2']
    wsk, bsk = P['skip0']

    def body(x_r, dm_r, w1_r, b1_r, w2_r, b2_r, wsk_r, bsk_r, x0_r, sk_r):
        x = x_r[...]
        bad = x[:, 0:1] == -999.0
        x = jnp.where(bad, dm_r[...], x)
        x0 = jax.nn.relu(x @ w1_r[...] + b1_r[...]) @ w2_r[...] + b2_r[...]
        x0_r[...] = jnp.concatenate(
            [x0, jnp.zeros((x0.shape[0], NODE_IN - HID), jnp.float32)], 1)
        sk_r[...] = x0 @ wsk_r[...] + bsk_r[...]

    return _tc(
        body, (GN,),
        [_rows(TN, NODE_IN), _full((1, NODE_IN)),
         _full((NODE_IN, HID)), _full((1, HID)),
         _full((HID, HID)), _full((1, HID)),
         _full((HID, OUT)), _full((1, OUT))],
        [_rows(TN, NODE_IN), _rows(TN, OUT)],
        [jax.ShapeDtypeStruct((N, NODE_IN), jnp.float32),
         jax.ShapeDtypeStruct((N, OUT), jnp.float32)],
    )(x_in, _r2(P['dummy']), wne1, _r2(bne1), wne2, _r2(bne2),
      wsk, _r2(bsk))


def _edge_embed(ea_s, P):
    w1, b1 = P['ee1']
    w2, b2 = P['ee2']
    EIN = int(w1.shape[0])

    def body(e_r, w1_r, b1_r, w2_r, b2_r, o_r):
        o_r[...] = (jax.nn.relu(e_r[...] @ w1_r[...] + b1_r[...])
                    @ w2_r[...] + b2_r[...])

    return _tc(
        body, (GE,),
        [_rows(TE, EIN), _full((EIN, HID)), _full((1, HID)),
         _full((HID, HID)), _full((1, HID))],
        [_rows(TE, HID)],
        [jax.ShapeDtypeStruct((EPAD, HID), jnp.float32)],
    )(ea_s, w1, _r2(b1), w2, _r2(b2))[0]


def _conv1_stage(xd, xs, e, cp):
    wa, ba1 = cp['a1']
    wa2, ba2 = cp['a2']
    wm1, bm1 = cp['m1']
    wm2, bm2 = cp['m2']

    def body(xd_r, xs_r, e_r, wad_r, was_r, wae_r, ba1_r, wa2_r, ba2_r,
             wmx_r, wme_r, bm1_r, wm2_r, bm2_r, wm_r, ex_r):
        xdv = xd_r[...][:, :HID]
        xsv = xs_r[...][:, :HID]
        e = e_r[...]
        h = jax.nn.relu(xdv @ wad_r[...] + xsv @ was_r[...]
                        + e @ wae_r[...] + ba1_r[...])
        ex = jnp.exp(h @ wa2_r[...] + ba2_r[...])
        mp = jax.nn.relu(xsv @ wmx_r[...] + e @ wme_r[...] + bm1_r[...])
        m = mp @ wm2_r[...] + bm2_r[...]
        parts = [m[:, c * HID:(c + 1) * HID] * ex[:, c:c + 1]
                 for c in range(HEADS)]
        wm_r[...] = jnp.concatenate(parts, axis=1)
        ex_r[...] = ex

    return _tc(
        body, (GE,),
        [_rows(TE, OUT), _rows(TE, OUT), _rows(TE, HID),
         _full((HID, HID)), _full((HID, HID)), _full((HID, HID)),
         _full((1, HID)), _full((HID, HEADS)), _full((1, HEADS)),
         _full((HID, OUT)), _full((HID, OUT)), _full((1, OUT)),
         _full((OUT, OUT)), _full((1, OUT))],
        [_rows(TE, OUT), _rows(TE, HEADS)],
        [jax.ShapeDtypeStruct((EPAD, OUT), jnp.float32),
         jax.ShapeDtypeStruct((EPAD, HEADS), jnp.float32)],
    )(xd, xs, e, wa[:HID], wa[HID:2 * HID], wa[2 * HID:], _r2(ba1),
      wa2, _r2(ba2), wm1[:HID], wm1[HID:], _r2(bm1), wm2, _r2(bm2))


def _pair_stage(xd, xs, e, up, cp, pad_e_out):
    w1, b1 = up['w1']
    w2, b2 = up['w2']
    lg, lb = up['ln']
    wa, ba1 = cp['a1']
    wa2, ba2 = cp['a2']
    wm1, bm1 = cp['m1']
    wm2, bm2 = cp['m2']
    ew = OUT if pad_e_out else HID

    def body(xd_r, xs_r, e_r, wud_r, wus_r, wue_r, b1_r, w2_r, b2_r,
             lg_r, lb_r, wad_r, was_r, wae_r, ba1_r, wa2_r, ba2_r,
             wmx_r, wme_r, bm1_r, wm2_r, bm2_r, en_r, wm_r, ex_r):
        xdv = xd_r[...]
        xsv = xs_r[...]
        e = e_r[...]
        d = jax.nn.relu(xdv @ wud_r[...] + xsv @ wus_r[...]
                        + e @ wue_r[...] + b1_r[...]) @ w2_r[...] + b2_r[...]
        en = _ln_in(e + d, lg_r[...], lb_r[...])
        if pad_e_out:
            en_r[...] = jnp.concatenate(
                [en, jnp.zeros((en.shape[0], OUT - HID), jnp.float32)], 1)
        else:
            en_r[...] = en
        h = jax.nn.relu(xdv @ wad_r[...] + xsv @ was_r[...]
                        + en @ wae_r[...] + ba1_r[...])
        ex = jnp.exp(h @ wa2_r[...] + ba2_r[...])
        mp = jax.nn.relu(xsv @ wmx_r[...] + en @ wme_r[...] + bm1_r[...])
        m = mp @ wm2_r[...] + bm2_r[...]
        parts = [m[:, c * HID:(c + 1) * HID] * ex[:, c:c + 1]
                 for c in range(HEADS)]
        wm_r[...] = jnp.concatenate(parts, axis=1)
        ex_r[...] = ex

    return _tc(
        body, (GE,),
        [_rows(TE, OUT), _rows(TE, OUT), _rows(TE, HID),
         _full((OUT, HID)), _full((OUT, HID)), _full((HID, HID)),
         _full((1, HID)), _full((HID, HID)), _full((1, HID)),
         _full((1, HID)), _full((1, HID)),
         _full((OUT, HID)), _full((OUT, HID)), _full((HID, HID)),
         _full((1, HID)), _full((HID, HEADS)), _full((1, HEADS)),
         _full((OUT, OUT)), _full((HID, OUT)), _full((1, OUT)),
         _full((OUT, OUT)), _full((1, OUT))],
        [_rows(TE, ew), _rows(TE, OUT), _rows(TE, HEADS)],
        [jax.ShapeDtypeStruct((EPAD, ew), jnp.float32),
         jax.ShapeDtypeStruct((EPAD, OUT), jnp.float32),
         jax.ShapeDtypeStruct((EPAD, HEADS), jnp.float32)],
    )(xd, xs, e, w1[:OUT], w1[OUT:2 * OUT], w1[2 * OUT:], _r2(b1),
      w2, _r2(b2), _r2(lg), _r2(lb),
      wa[:OUT], wa[OUT:2 * OUT], wa[2 * OUT:], _r2(ba1), wa2, _r2(ba2),
      wm1[:OUT], wm1[OUT:], _r2(bm1), wm2, _r2(bm2))


def _combine(agg, den, skip, lnp):
    g, b = lnp

    def body(agg_r, den_r, sk_r, g_r, b_r, x_r):
        den = den_r[...]
        parts = [agg_r[...][:, c * HID:(c + 1) * HID] /
                 (den[:, c:c + 1] + 1e-16) for c in range(HEADS)]
        x = jnp.concatenate(parts, axis=1) + sk_r[...]
        x_r[...] = jax.nn.relu(_ln_in(x, g_r[...], b_r[...]))

    return _tc(
        body, (GN,),
        [_rows(TN, OUT), _rows(TN, 16), _rows(TN, OUT),
         _full((1, OUT)), _full((1, OUT))],
        [_rows(TN, OUT)],
        [jax.ShapeDtypeStruct((N, OUT), jnp.float32)],
    )(agg, den, skip, _r2(g), _r2(b))[0]


def _final_node(agg, den, x2, x1, P):
    g3, b3 = P['ln3']
    wsl, bsl = P['skipL']
    w1, c1 = P['nh1']
    w2, c2 = P['nh2']
    w3, c3 = P['nh3']
    w4, c4 = P['nh4']
    we1 = P['eh1'][0]
    whs, whd = we1[:OUT], we1[OUT:2 * OUT]
    NCLS = int(w4.shape[1])

    def body(agg_r, den_r, x2_r, x1_r, g_r, b_r, wsl_r, bsl_r,
             w1_r, c1_r, w2_r, c2_r, w3_r, c3_r, w4_r, c4_r,
             whd_r, whs_r, lg_r, pr_r, hd_r, hs_r):
        den = den_r[...]
        parts = [agg_r[...][:, c * HID:(c + 1) * HID] /
                 (den[:, c:c + 1] + 1e-16) for c in range(HEADS)]
        x3 = jnp.concatenate(parts, axis=1) + x2_r[...]
        x3 = jax.nn.relu(_ln_in(x3, g_r[...], b_r[...]))
        xo = x3 + x1_r[...] @ wsl_r[...] + bsl_r[...]
        h = jax.nn.relu(xo @ w1_r[...] + c1_r[...])
        h = jax.nn.relu(h @ w2_r[...] + c2_r[...])
        h = jax.nn.relu(h @ w3_r[...] + c3_r[...])
        lg = h @ w4_r[...] + c4_r[...]
        lg_r[...] = lg
        mx = jnp.max(lg, -1, keepdims=True)
        p = jnp.exp(lg - mx)
        pr_r[...] = p / jnp.sum(p, -1, keepdims=True)
        hd_r[...] = xo @ whd_r[...]
        hs_r[...] = xo @ whs_r[...]

    return _tc(
        body, (GN,),
        [_rows(TN, OUT), _rows(TN, 16), _rows(TN, OUT), _rows(TN, OUT),
         _full((1, OUT)), _full((1, OUT)), _full((OUT, OUT)),
         _full((1, OUT)), _full((OUT, 64)), _full((1, 64)),
         _full((64, 32)), _full((1, 32)), _full((32, 16)), _full((1, 16)),
         _full((16, NCLS)), _full((1, NCLS)),
         _full((OUT, OUT)), _full((OUT, OUT))],
        [_rows(TN, NCLS), _rows(TN, NCLS), _rows(TN, OUT), _rows(TN, OUT)],
        [jax.ShapeDtypeStruct((N, NCLS), jnp.float32),
         jax.ShapeDtypeStruct((N, NCLS), jnp.float32),
         jax.ShapeDtypeStruct((N, OUT), jnp.float32),
         jax.ShapeDtypeStruct((N, OUT), jnp.float32)],
    )(agg, den, x2, x1, _r2(g3), _r2(b3), wsl, _r2(bsl),
      w1, _r2(c1), w2, _r2(c2), w3, _r2(c3), w4, _r2(c4), whd, whs)


def _edge_head(ghd, ghs, e2o, P):
    we1, b1 = P['eh1']
    w2, b2 = P['eh2']
    w3, b3 = P['eh3']
    w4, b4 = P['eh4']
    whe = we1[2 * OUT:]

    def body(gd_r, gs_r, e_r, we_r, b1_r, w2_r, b2_r, w3_r, b3_r,
             w4_r, b4_r, l_r, p_r):
        g = jax.nn.relu(gd_r[...] + gs_r[...]
                        + e_r[...][:, :HID] @ we_r[...] + b1_r[...])
        g = jax.nn.relu(g @ w2_r[...] + b2_r[...])
        g = jax.nn.relu(g @ w3_r[...] + b3_r[...])
        l = g @ w4_r[...] + b4_r[...]
        l_r[...] = l
        p_r[...] = jax.nn.sigmoid(l)

    return _tc(
        body, (GE,),
        [_rows(TE, OUT), _rows(TE, OUT), _rows(TE, OUT),
         _full((HID, OUT)), _full((1, OUT)), _full((OUT, 64)),
         _full((1, 64)), _full((64, 32)), _full((1, 32)),
         _full((32, 1)), _full((1, 1))],
        [_rows(TE, 1), _rows(TE, 1)],
        [jax.ShapeDtypeStruct((EPAD, 1), jnp.float32),
         jax.ShapeDtypeStruct((EPAD, 1), jnp.float32)],
    )(ghd, ghs, e2o, whe, _r2(b1), w2, _r2(b2), w3, _r2(b3), w4, _r2(b4))


def _conv_pass(dst_p, spans, wm, ex):
    aggf, denf = _sc_segsum(dst_p, wm.reshape(-1), ex.reshape(-1), spans)
    agg = aggf.reshape(NPAD, OUT)[:N]
    den = denf.reshape(NPAD, 16)[:N]
    return agg, den


def kernel(x_in, edge_index, edge_attr, params):
    P = params
    src_o, dst_o = edge_index[0], edge_index[1]
    iota = lax.iota(jnp.int32, E)
    dst_s, perm = lax.sort_key_val(dst_o, iota)
    src_s = jnp.take(src_o, perm)
    ea_s = jnp.take(edge_attr, perm, axis=0)
    ea_p = jnp.concatenate(
        [ea_s, jnp.zeros((EPAD - E, ea_s.shape[1]), jnp.float32)])

    padi = jnp.zeros((EPAD - E,), jnp.int32)
    dst_p = jnp.concatenate([dst_s, padi])
    src_p = jnp.concatenate([src_s, padi])
    dsto_p = jnp.concatenate([dst_o, padi])
    srco_p = jnp.concatenate([src_o, padi])
    perm_p = jnp.concatenate([perm, jnp.arange(E, EPAD, dtype=jnp.int32)])

    bounds = jnp.arange(33, dtype=jnp.int32) * RPW
    starts = jnp.searchsorted(dst_s, bounds).astype(jnp.int32)
    spans = jnp.pad(jnp.stack([starts[:-1], starts[1:]], axis=1),
                    ((0, 0), (0, 126)))

    x0p, sk0 = _node_pre(x_in, P)
    e0 = _edge_embed(ea_p, P)

    xd0, xs0 = _sc_gather2(x0p, x0p, dst_p, src_p)
    wm1, ex1 = _conv1_stage(xd0, xs0, e0, P['conv1'])
    agg1, den1 = _conv_pass(dst_p, spans, wm1, ex1)
    x1 = _combine(agg1, den1, sk0, P['ln1'])

    xd1, xs1 = _sc_gather2(x1, x1, dst_p, src_p)
    e1, wm2, ex2 = _pair_stage(xd1, xs1, e0, P['eup1'], P['conv2'], False)
    agg2, den2 = _conv_pass(dst_p, spans, wm2, ex2)
    x2 = _combine(agg2, den2, x1, P['ln2'])

    xd2, xs2 = _sc_gather2(x2, x2, dst_p, src_p)
    e2s, wm3, ex3 = _pair_stage(xd2, xs2, e1, P['eup2'], P['conv3'], True)
    agg3, den3 = _conv_pass(dst_p, spans, wm3, ex3)

    node_logits, node_probs, hd, hs = _final_node(agg3, den3, x2, x1, P)

    ghd, ghs, e2o = _sc_gather2(hd, hs, dsto_p, srco_p,
                                scat_vals=e2s, scat_idx=perm_p)
    el, ep = _edge_head(ghd, ghs, e2o, P)

    return (node_logits, el[:E], node_probs, ep[:E])
